# Initial kernel scaffold; baseline (speedup 1.0000x reference)
import jax
import jax.numpy as jnp
from jax.experimental import pallas as pl


def kernel(node_feats, adj_matrix, W, b, a):
    raise NotImplementedError("write your pallas kernel here")


# speedup vs baseline: 665.8246x; 665.8246x over previous
import functools

import jax
import jax.numpy as jnp
from jax.experimental import pallas as pl
from jax.experimental.pallas import tpu as pltpu

_ALPHA = 0.2
_NEG = -9e15


def _gat_block_kernel(cr_ref, x_ref, xT_ref, adj_ref, Wt_ref, b_ref, al_ref,
                      wr_ref, out_ref, nf_scr, sdst_scr, *, block_rows):
    i = pl.program_id(0)

    @pl.when(i == 0)
    def _init():
        nf = jnp.dot(x_ref[...], Wt_ref[...],
                     preferred_element_type=jnp.float32) + b_ref[...]
        nf_scr[...] = nf
        sdst_scr[...] = jnp.dot(wr_ref[...], xT_ref[...],
                                preferred_element_type=jnp.float32) + cr_ref[0]

    nf = nf_scr[...]
    nf_rows = nf_scr[pl.ds(i * block_rows, block_rows), :]
    s_src = jnp.dot(nf_rows, al_ref[...],
                    preferred_element_type=jnp.float32)
    logits = s_src + sdst_scr[...]
    logits = jnp.where(logits > 0, logits, _ALPHA * logits)
    masked = jnp.where(adj_ref[...] == 1, logits, _NEG)
    m = jnp.max(masked, axis=1, keepdims=True)
    e = jnp.exp(masked - m)
    s = jnp.sum(e, axis=1, keepdims=True)
    probs = e / s
    out_ref[...] = jnp.dot(probs, nf, preferred_element_type=jnp.float32)


@jax.jit
def kernel(node_feats, adj_matrix, W, b, a):
    batch, n, c_in = node_feats.shape
    c_out = W.shape[0]
    x = node_feats.reshape(n, c_in)
    xT = x.T
    adj = adj_matrix.reshape(n, n)
    Wt = W.T
    a_l = a[0, :c_out].reshape(c_out, 1)
    a_r = a[0, c_out:]
    wr = (Wt @ a_r).reshape(1, c_in)
    cr = jnp.dot(b, a_r).reshape(1)
    b_row = b.reshape(1, c_out)

    block_rows = 128
    grid = n // block_rows

    out = pl.pallas_call(
        functools.partial(_gat_block_kernel, block_rows=block_rows),
        grid_spec=pltpu.PrefetchScalarGridSpec(
            num_scalar_prefetch=0,
            grid=(grid,),
            in_specs=[
                pl.BlockSpec(memory_space=pltpu.SMEM),
                pl.BlockSpec((n, c_in), lambda i: (0, 0)),
                pl.BlockSpec((c_in, n), lambda i: (0, 0)),
                pl.BlockSpec((block_rows, n), lambda i: (i, 0)),
                pl.BlockSpec((c_in, c_out), lambda i: (0, 0)),
                pl.BlockSpec((1, c_out), lambda i: (0, 0)),
                pl.BlockSpec((c_out, 1), lambda i: (0, 0)),
                pl.BlockSpec((1, c_in), lambda i: (0, 0)),
            ],
            out_specs=pl.BlockSpec((block_rows, c_out), lambda i: (i, 0)),
            scratch_shapes=[
                pltpu.VMEM((n, c_out), jnp.float32),
                pltpu.VMEM((1, n), jnp.float32),
            ],
        ),
        out_shape=jax.ShapeDtypeStruct((n, c_out), jnp.float32),
        compiler_params=pltpu.CompilerParams(
            dimension_semantics=("arbitrary",),
        ),
    )(cr, x, xT, adj, Wt, b_row, a_l, wr)

    return out.reshape(batch, n, c_out)


# speedup vs baseline: 1222.9319x; 1.8367x over previous
import functools

import jax
import jax.numpy as jnp
from jax import lax
from jax.experimental import pallas as pl
from jax.experimental.pallas import tpu as pltpu

_ALPHA = 0.2
_NEG = -9e15
_NT = (((1,), (1,)), ((), ()))


def _gat_block_kernel(x_ref, adj_ref, w_ref, b_ref, a_ref, out_ref,
                      nf_scr, sdst_scr, *, block_rows, c_out):
    i = pl.program_id(0)

    @pl.when(i == 0)
    def _init():
        nf = lax.dot_general(x_ref[...], w_ref[...], _NT,
                             preferred_element_type=jnp.float32)
        nf = nf + b_ref[...]
        nf_scr[...] = nf
        sdst_scr[...] = lax.dot_general(a_ref[:, c_out:], nf, _NT,
                                        preferred_element_type=jnp.float32)

    nf = nf_scr[...]
    nf_rows = nf_scr[pl.ds(i * block_rows, block_rows), :]
    s_src = lax.dot_general(nf_rows, a_ref[:, :c_out], _NT,
                            preferred_element_type=jnp.float32)
    logits = s_src + sdst_scr[...]
    logits = jnp.where(logits > 0, logits, _ALPHA * logits)
    masked = jnp.where(adj_ref[...] == 1, logits, _NEG)
    m = jnp.max(masked, axis=1, keepdims=True)
    e = jnp.exp(masked - m)
    s = jnp.sum(e, axis=1, keepdims=True)
    agg = jnp.dot(e, nf, preferred_element_type=jnp.float32)
    out_ref[...] = agg / s


@jax.jit
def kernel(node_feats, adj_matrix, W, b, a):
    batch, n, c_in = node_feats.shape
    c_out = W.shape[0]
    x = node_feats.reshape(n, c_in)
    adj = adj_matrix.reshape(n, n)
    b_row = b.reshape(1, c_out)

    block_rows = 128
    grid = n // block_rows

    out = pl.pallas_call(
        functools.partial(_gat_block_kernel, block_rows=block_rows,
                          c_out=c_out),
        grid=(grid,),
        in_specs=[
            pl.BlockSpec((n, c_in), lambda i: (0, 0)),
            pl.BlockSpec((block_rows, n), lambda i: (i, 0)),
            pl.BlockSpec((c_out, c_in), lambda i: (0, 0)),
            pl.BlockSpec((1, c_out), lambda i: (0, 0)),
            pl.BlockSpec((1, 2 * c_out), lambda i: (0, 0)),
        ],
        out_specs=pl.BlockSpec((block_rows, c_out), lambda i: (i, 0)),
        scratch_shapes=[
            pltpu.VMEM((n, c_out), jnp.float32),
            pltpu.VMEM((1, n), jnp.float32),
        ],
        out_shape=jax.ShapeDtypeStruct((n, c_out), jnp.float32),
        compiler_params=pltpu.CompilerParams(
            dimension_semantics=("arbitrary",),
        ),
    )(x, adj, W, b_row, a)

    return out.reshape(batch, n, c_out)


# speedup vs baseline: 1551.3458x; 1.2685x over previous
import functools

import jax
import jax.numpy as jnp
from jax import lax
from jax.experimental import pallas as pl
from jax.experimental.pallas import tpu as pltpu

_ALPHA = 0.2
_NEG = -9e15
_NT = (((1,), (1,)), ((), ()))


def _gat_block_kernel(x_ref, adj_ref, w_ref, b_ref, a_ref, out_ref,
                      nf_scr, sdst_scr, *, block_rows, c_out):
    i = pl.program_id(0)

    @pl.when(i == 0)
    def _init():
        nf = lax.dot_general(x_ref[...], w_ref[...], _NT,
                             preferred_element_type=jnp.float32)
        nf = nf + b_ref[...]
        nf_scr[...] = nf
        sdst_scr[...] = lax.dot_general(a_ref[:, c_out:], nf, _NT,
                                        preferred_element_type=jnp.float32)

    nf = nf_scr[...]
    nf_rows = nf_scr[pl.ds(i * block_rows, block_rows), :]
    s_src = lax.dot_general(nf_rows, a_ref[:, :c_out], _NT,
                            preferred_element_type=jnp.float32)
    logits = s_src + sdst_scr[...]
    logits = jnp.where(logits > 0, logits, _ALPHA * logits)
    masked = jnp.where(adj_ref[...] == 1, logits, _NEG)
    m = jnp.max(masked, axis=1, keepdims=True)
    e = jnp.exp(masked - m)
    s = jnp.sum(e, axis=1, keepdims=True)
    agg = jnp.dot(e, nf, preferred_element_type=jnp.float32)
    out_ref[...] = agg / s


@jax.jit
def kernel(node_feats, adj_matrix, W, b, a):
    batch, n, c_in = node_feats.shape
    c_out = W.shape[0]
    x = node_feats.reshape(n, c_in)
    adj = adj_matrix.reshape(n, n)
    b_row = b.reshape(1, c_out)

    block_rows = 256
    grid = n // block_rows

    out = pl.pallas_call(
        functools.partial(_gat_block_kernel, block_rows=block_rows,
                          c_out=c_out),
        grid=(grid,),
        in_specs=[
            pl.BlockSpec((n, c_in), lambda i: (0, 0)),
            pl.BlockSpec((block_rows, n), lambda i: (i, 0)),
            pl.BlockSpec((c_out, c_in), lambda i: (0, 0)),
            pl.BlockSpec((1, c_out), lambda i: (0, 0)),
            pl.BlockSpec((1, 2 * c_out), lambda i: (0, 0)),
        ],
        out_specs=pl.BlockSpec((block_rows, c_out), lambda i: (i, 0)),
        scratch_shapes=[
            pltpu.VMEM((n, c_out), jnp.float32),
            pltpu.VMEM((1, n), jnp.float32),
        ],
        out_shape=jax.ShapeDtypeStruct((n, c_out), jnp.float32),
        compiler_params=pltpu.CompilerParams(
            dimension_semantics=("arbitrary",),
        ),
    )(x, adj, W, b_row, a)

    return out.reshape(batch, n, c_out)


# speedup vs baseline: 1739.0184x; 1.1210x over previous
import functools

import jax
import jax.numpy as jnp
from jax import lax
from jax.experimental import pallas as pl
from jax.experimental.pallas import tpu as pltpu

_ALPHA = 0.2
_NEG = -9e15
_NT = (((1,), (1,)), ((), ()))


def _gat_block_kernel(x_ref, adj_ref, w_ref, b_ref, a_ref, out_ref,
                      nf_scr, sdst_scr, *, block_rows, c_out):
    i = pl.program_id(0)

    @pl.when(i == 0)
    def _init():
        nf = lax.dot_general(x_ref[...], w_ref[...], _NT,
                             preferred_element_type=jnp.float32)
        nf = nf + b_ref[...]
        nf_scr[...] = nf
        sdst_scr[...] = lax.dot_general(a_ref[:, c_out:], nf, _NT,
                                        preferred_element_type=jnp.float32)

    nf = nf_scr[...]
    nf_rows = nf_scr[pl.ds(i * block_rows, block_rows), :]
    s_src = lax.dot_general(nf_rows, a_ref[:, :c_out], _NT,
                            preferred_element_type=jnp.float32)
    logits = s_src + sdst_scr[...]
    logits = jnp.where(logits > 0, logits, _ALPHA * logits)
    masked = jnp.where(adj_ref[...] == 1, logits, _NEG)
    m = jnp.max(masked, axis=1, keepdims=True)
    e = jnp.exp(masked - m)
    s = jnp.sum(e, axis=1, keepdims=True)
    agg = jnp.dot(e, nf, preferred_element_type=jnp.float32)
    out_ref[...] = agg / s


@jax.jit
def kernel(node_feats, adj_matrix, W, b, a):
    batch, n, c_in = node_feats.shape
    c_out = W.shape[0]
    x = node_feats.reshape(n, c_in)
    adj = adj_matrix.reshape(n, n)
    b_row = b.reshape(1, c_out)

    block_rows = 512
    grid = n // block_rows

    out = pl.pallas_call(
        functools.partial(_gat_block_kernel, block_rows=block_rows,
                          c_out=c_out),
        grid=(grid,),
        in_specs=[
            pl.BlockSpec((n, c_in), lambda i: (0, 0)),
            pl.BlockSpec((block_rows, n), lambda i: (i, 0)),
            pl.BlockSpec((c_out, c_in), lambda i: (0, 0)),
            pl.BlockSpec((1, c_out), lambda i: (0, 0)),
            pl.BlockSpec((1, 2 * c_out), lambda i: (0, 0)),
        ],
        out_specs=pl.BlockSpec((block_rows, c_out), lambda i: (i, 0)),
        scratch_shapes=[
            pltpu.VMEM((n, c_out), jnp.float32),
            pltpu.VMEM((1, n), jnp.float32),
        ],
        out_shape=jax.ShapeDtypeStruct((n, c_out), jnp.float32),
        compiler_params=pltpu.CompilerParams(
            dimension_semantics=("arbitrary",),
        ),
    )(x, adj, W, b_row, a)

    return out.reshape(batch, n, c_out)
